# Initial kernel scaffold; baseline (speedup 1.0000x reference)
import jax
import jax.numpy as jnp
from jax.experimental import pallas as pl


def kernel(ui_uKey, ui_iKey, uEmbeds, iEmbeds, ui_uHyper, ui_iHyper, rows, cols, edgeids):
    raise NotImplementedError("write your pallas kernel here")


# speedup vs baseline: 2.1346x; 2.1346x over previous
import functools

import jax
import jax.numpy as jnp
from jax import lax
from jax.experimental import pallas as pl
from jax.experimental.pallas import tpu as pltpu
from jax.experimental.pallas import tpu_sc as plsc

N_USERS = 50000
LATDIM = 64
DFUSED = 128
E_TOTAL = 1600000

NC, NS, LANES = 2, 16, 16
NW = NC * NS
CHUNK = 128
N_CHUNKS = E_TOTAL // CHUNK
BASE_CHUNKS = N_CHUNKS // NW
EXTRA_CHUNKS = N_CHUNKS % NW
ROW_BLK = 2000


def _prep_body(ukey_ref, uemb_ref, ikey_ref, iemb_ref, uh_ref, ih_ref,
               a_ref, b_ref):
    m = lax.dot_general(uh_ref[...], ih_ref[...], (((1,), (1,)), ((), ())),
                        preferred_element_type=jnp.float32,
                        precision=lax.Precision.HIGHEST)
    a_ref[:, 0:LATDIM] = lax.dot_general(
        ukey_ref[...], m, (((1,), (0,)), ((), ())),
        preferred_element_type=jnp.float32, precision=lax.Precision.HIGHEST)
    a_ref[:, LATDIM:DFUSED] = uemb_ref[...]
    b_ref[:, 0:LATDIM] = ikey_ref[...]
    b_ref[:, LATDIM:DFUSED] = iemb_ref[...]


def _prep_tables(ukey, uemb, ikey, iemb, uh, ih):
    grid = N_USERS // ROW_BLK
    row_spec = pl.BlockSpec((ROW_BLK, LATDIM), lambda i: (i, 0))
    hyper_spec = pl.BlockSpec((LATDIM, 128), lambda i: (0, 0))
    out_spec = pl.BlockSpec((ROW_BLK, DFUSED), lambda i: (i, 0))
    return pl.pallas_call(
        _prep_body,
        grid=(grid,),
        in_specs=[row_spec, row_spec, row_spec, row_spec, hyper_spec, hyper_spec],
        out_specs=[out_spec, out_spec],
        out_shape=[
            jax.ShapeDtypeStruct((N_USERS, DFUSED), jnp.float32),
            jax.ShapeDtypeStruct((N_USERS, DFUSED), jnp.float32),
        ],
    )(ukey, uemb, ikey, iemb, uh, ih)


def _sc_body(a_hbm, b_hbm, rows_hbm, cols_hbm, eids_hbm, out_hbm,
             eids_v, u_v, i_v, arows_v, brows_v, out_v, sem):
    cid = lax.axis_index("c")
    sid = lax.axis_index("s")
    wid = sid * NC + cid
    my_n = BASE_CHUNKS + jnp.where(wid < EXTRA_CHUNKS, 1, 0)

    def chunk_body(k, _):
        c = wid + k * NW
        off = c * CHUNK
        pltpu.sync_copy(eids_hbm.at[pl.ds(off, CHUNK)], eids_v)
        cp_u = pltpu.async_copy(rows_hbm.at[eids_v], u_v, sem)
        cp_i = pltpu.async_copy(cols_hbm.at[eids_v], i_v, sem)
        cp_u.wait()
        cp_i.wait()
        cp_a = pltpu.async_copy(a_hbm.at[u_v], arows_v, sem)
        cp_b = pltpu.async_copy(b_hbm.at[i_v], brows_v, sem)
        cp_a.wait()
        cp_b.wait()
        for g in range(CHUNK // LANES):
            rowv = lax.iota(jnp.int32, LANES) + g * LANES

            def d_body(d, acc):
                cv = jnp.full((LANES,), d, dtype=jnp.int32)
                ga = plsc.load_gather(arows_v, [rowv, cv])
                gb = plsc.load_gather(brows_v, [rowv, cv])
                return acc + ga * gb

            zero = jnp.zeros((LANES,), jnp.float32)
            acc_s = lax.fori_loop(0, LATDIM, d_body, zero, unroll=8)
            acc_p = lax.fori_loop(LATDIM, DFUSED, d_body, zero, unroll=8)
            score = 1.0 / (1.0 + jnp.exp(-acc_s))
            out_v[pl.ds(g * LANES, LANES)] = jnp.abs(score - acc_p)
        pltpu.sync_copy(out_v, out_hbm.at[pl.ds(off, CHUNK)])
        return _

    lax.fori_loop(0, my_n, chunk_body, None)


def _edge_scores(a_tab, b_tab, rows, cols, edgeids):
    mesh = plsc.VectorSubcoreMesh(core_axis_name="c", subcore_axis_name="s",
                                  num_cores=NC, num_subcores=NS)
    f = pl.kernel(
        _sc_body,
        out_type=jax.ShapeDtypeStruct((E_TOTAL,), jnp.float32),
        mesh=mesh,
        compiler_params=pltpu.CompilerParams(needs_layout_passes=False),
        scratch_types=[
            pltpu.VMEM((CHUNK,), jnp.int32),
            pltpu.VMEM((CHUNK,), jnp.int32),
            pltpu.VMEM((CHUNK,), jnp.int32),
            pltpu.VMEM((CHUNK, DFUSED), jnp.float32),
            pltpu.VMEM((CHUNK, DFUSED), jnp.float32),
            pltpu.VMEM((CHUNK,), jnp.float32),
            pltpu.SemaphoreType.DMA,
        ],
    )
    return f(a_tab, b_tab, rows, cols, edgeids)


def kernel(ui_uKey, ui_iKey, uEmbeds, iEmbeds, ui_uHyper, ui_iHyper,
           rows, cols, edgeids):
    ukey = jnp.transpose(ui_uKey, (1, 0, 2)).reshape(-1, LATDIM)
    ikey = jnp.transpose(ui_iKey, (1, 0, 2)).reshape(-1, LATDIM)
    a_tab, b_tab = _prep_tables(ukey, uEmbeds, ikey, iEmbeds,
                                ui_uHyper, ui_iHyper)
    return _edge_scores(a_tab, b_tab, rows, cols, edgeids)


# speedup vs baseline: 2.5914x; 1.2140x over previous
import jax
import jax.numpy as jnp
from jax import lax
from jax.experimental import pallas as pl
from jax.experimental.pallas import tpu as pltpu
from jax.experimental.pallas import tpu_sc as plsc

N_USERS = 50000
LATDIM = 64
DFUSED = 128
E_TOTAL = 1600000

NC, NS, LANES = 2, 16, 16
NW = NC * NS
CHUNK = 128
N_CHUNKS = E_TOTAL // CHUNK
BASE_CHUNKS = N_CHUNKS // NW
EXTRA_CHUNKS = N_CHUNKS % NW
ROW_BLK = 2000


def _prep_body(ukey_ref, uemb_ref, ikey_ref, iemb_ref, uh_ref, ih_ref,
               a_ref, b_ref):
    m = lax.dot_general(uh_ref[...], ih_ref[...], (((1,), (1,)), ((), ())),
                        preferred_element_type=jnp.float32,
                        precision=lax.Precision.HIGHEST)
    a_ref[:, 0:LATDIM] = lax.dot_general(
        ukey_ref[...], m, (((1,), (0,)), ((), ())),
        preferred_element_type=jnp.float32, precision=lax.Precision.HIGHEST)
    a_ref[:, LATDIM:DFUSED] = uemb_ref[...]
    b_ref[:, 0:LATDIM] = ikey_ref[...]
    b_ref[:, LATDIM:DFUSED] = iemb_ref[...]


def _prep_tables(ukey, uemb, ikey, iemb, uh, ih):
    grid = N_USERS // ROW_BLK
    row_spec = pl.BlockSpec((ROW_BLK, LATDIM), lambda i: (i, 0))
    hyper_spec = pl.BlockSpec((LATDIM, 128), lambda i: (0, 0))
    out_spec = pl.BlockSpec((ROW_BLK, DFUSED), lambda i: (i, 0))
    return pl.pallas_call(
        _prep_body,
        grid=(grid,),
        in_specs=[row_spec, row_spec, row_spec, row_spec, hyper_spec, hyper_spec],
        out_specs=[out_spec, out_spec],
        out_shape=[
            jax.ShapeDtypeStruct((N_USERS, DFUSED), jnp.float32),
            jax.ShapeDtypeStruct((N_USERS, DFUSED), jnp.float32),
        ],
    )(ukey, uemb, ikey, iemb, uh, ih)


def _sc_body(a_hbm, b_hbm, rows_hbm, cols_hbm, eids_hbm, out_hbm,
             eids_v, u_v, i_v, a_v, b_v, out_v,
             sem_e0, sem_e1, sem_ui0, sem_ui1, sem_ab0, sem_ab1):
    cid = lax.axis_index("c")
    sid = lax.axis_index("s")
    wid = sid * NC + cid
    my_n = BASE_CHUNKS + jnp.where(wid < EXTRA_CHUNKS, 1, 0)
    sem_e = (sem_e0, sem_e1)
    sem_ui = (sem_ui0, sem_ui1)
    sem_ab = (sem_ab0, sem_ab1)

    def goff(j):
        return (wid + j * NW) * CHUNK

    def eids_slot(s):
        return eids_v.at[s]

    def issue_eids(j, s):
        return pltpu.async_copy(eids_hbm.at[pl.ds(goff(j), CHUNK)],
                                eids_slot(s), sem_e[s])

    def issue_ui(s):
        cu = pltpu.async_copy(rows_hbm.at[eids_slot(s)], u_v.at[s], sem_ui[s])
        ci = pltpu.async_copy(cols_hbm.at[eids_slot(s)], i_v.at[s], sem_ui[s])
        return cu, ci

    def issue_ab(s):
        ca = pltpu.async_copy(a_hbm.at[u_v.at[s]], a_v.at[s], sem_ab[s])
        cb = pltpu.async_copy(b_hbm.at[i_v.at[s]], b_v.at[s], sem_ab[s])
        return ca, cb

    def wait_eids(s):
        pltpu.make_async_copy(eids_hbm.at[pl.ds(0, CHUNK)], eids_slot(s),
                              sem_e[s]).wait()

    def wait_ui(s):
        pltpu.make_async_copy(rows_hbm.at[pl.ds(0, CHUNK)], u_v.at[s],
                              sem_ui[s]).wait()
        pltpu.make_async_copy(cols_hbm.at[pl.ds(0, CHUNK)], i_v.at[s],
                              sem_ui[s]).wait()

    def wait_ab(s):
        pltpu.make_async_copy(a_hbm.at[pl.ds(0, CHUNK)], a_v.at[s],
                              sem_ab[s]).wait()
        pltpu.make_async_copy(b_hbm.at[pl.ds(0, CHUNK)], b_v.at[s],
                              sem_ab[s]).wait()

    def compute_chunk(k, s):
        for g in range(CHUNK // LANES):
            rowv = lax.iota(jnp.int32, LANES) + g * LANES

            def d_body(d, carry):
                acc, cv = carry
                ga = plsc.load_gather(a_v.at[s], [rowv, cv])
                gb = plsc.load_gather(b_v.at[s], [rowv, cv])
                return acc + ga * gb, cv + 1

            zero = jnp.zeros((LANES,), jnp.float32)
            cv0 = jnp.zeros((LANES,), jnp.int32)
            acc_s, cv64 = lax.fori_loop(0, LATDIM, d_body, (zero, cv0),
                                        unroll=16)
            acc_p, _ = lax.fori_loop(0, LATDIM, d_body, (zero, cv64),
                                     unroll=16)
            score = 1.0 / (1.0 + jnp.exp(-acc_s))
            out_v[pl.ds(g * LANES, LANES)] = jnp.abs(score - acc_p)
        pltpu.sync_copy(out_v, out_hbm.at[pl.ds(goff(k), CHUNK)])

    def step(k, s):
        t = 1 - s
        wait_ab(s)

        @pl.when(k + 2 < my_n)
        def _prefetch_ui():
            wait_eids(s)
            issue_ui(s)

        @pl.when(k + 1 < my_n)
        def _start_ab():
            wait_ui(t)
            issue_ab(t)

        @pl.when(k + 3 < my_n)
        def _prefetch_eids():
            issue_eids(k + 3, t)

        compute_chunk(k, s)

    issue_eids(0, 0).wait()
    issue_ui(0)
    issue_eids(1, 1).wait()
    wait_ui(0)
    issue_ab(0)
    issue_ui(1)
    issue_eids(2, 0)

    def pair_body(p, _):
        k0 = 2 * p
        step(k0, 0)

        @pl.when(k0 + 1 < my_n)
        def _odd():
            step(k0 + 1, 1)

        return _

    lax.fori_loop(0, (my_n + 1) // 2, pair_body, None)


def _edge_scores(a_tab, b_tab, rows, cols, edgeids):
    mesh = plsc.VectorSubcoreMesh(core_axis_name="c", subcore_axis_name="s",
                                  num_cores=NC, num_subcores=NS)
    f = pl.kernel(
        _sc_body,
        out_type=jax.ShapeDtypeStruct((E_TOTAL,), jnp.float32),
        mesh=mesh,
        compiler_params=pltpu.CompilerParams(needs_layout_passes=False),
        scratch_types=[
            pltpu.VMEM((2, CHUNK), jnp.int32),
            pltpu.VMEM((2, CHUNK), jnp.int32),
            pltpu.VMEM((2, CHUNK), jnp.int32),
            pltpu.VMEM((2, CHUNK, DFUSED), jnp.float32),
            pltpu.VMEM((2, CHUNK, DFUSED), jnp.float32),
            pltpu.VMEM((CHUNK,), jnp.float32),
            pltpu.SemaphoreType.DMA,
            pltpu.SemaphoreType.DMA,
            pltpu.SemaphoreType.DMA,
            pltpu.SemaphoreType.DMA,
            pltpu.SemaphoreType.DMA,
            pltpu.SemaphoreType.DMA,
        ],
    )
    return f(a_tab, b_tab, rows, cols, edgeids)


def kernel(ui_uKey, ui_iKey, uEmbeds, iEmbeds, ui_uHyper, ui_iHyper,
           rows, cols, edgeids):
    ukey = jnp.transpose(ui_uKey, (1, 0, 2)).reshape(-1, LATDIM)
    ikey = jnp.transpose(ui_iKey, (1, 0, 2)).reshape(-1, LATDIM)
    a_tab, b_tab = _prep_tables(ukey, uEmbeds, ikey, iEmbeds,
                                ui_uHyper, ui_iHyper)
    return _edge_scores(a_tab, b_tab, rows, cols, edgeids)


# speedup vs baseline: 13.1521x; 5.0753x over previous
import jax
import jax.numpy as jnp
from jax import lax
from jax.experimental import pallas as pl
from jax.experimental.pallas import tpu as pltpu
from jax.experimental.pallas import tpu_sc as plsc

N_USERS = 50000
LATDIM = 64
DFUSED = 128
E_TOTAL = 1600000

NC, NS, LANES = 2, 16, 16
NW = NC * NS
CHUNK = 128
N_CHUNKS = E_TOTAL // CHUNK
BASE_CHUNKS = N_CHUNKS // NW
EXTRA_CHUNKS = N_CHUNKS % NW
ROW_BLK = 2000


def _prep_body(ukey_ref, uemb_ref, ikey_ref, iemb_ref, uh_ref, ih_ref,
               a_ref, b_ref):
    m = lax.dot_general(uh_ref[...], ih_ref[...], (((1,), (1,)), ((), ())),
                        preferred_element_type=jnp.float32,
                        precision=lax.Precision.HIGHEST)
    a_ref[:, 0:LATDIM] = lax.dot_general(
        ukey_ref[...], m, (((1,), (0,)), ((), ())),
        preferred_element_type=jnp.float32, precision=lax.Precision.HIGHEST)
    a_ref[:, LATDIM:DFUSED] = uemb_ref[...]
    b_ref[:, 0:LATDIM] = ikey_ref[...]
    b_ref[:, LATDIM:DFUSED] = iemb_ref[...]


def _prep_tables(ukey, uemb, ikey, iemb, uh, ih):
    grid = N_USERS // ROW_BLK
    row_spec = pl.BlockSpec((ROW_BLK, LATDIM), lambda i: (i, 0))
    hyper_spec = pl.BlockSpec((LATDIM, 128), lambda i: (0, 0))
    out_spec = pl.BlockSpec((ROW_BLK, DFUSED), lambda i: (i, 0))
    return pl.pallas_call(
        _prep_body,
        grid=(grid,),
        in_specs=[row_spec, row_spec, row_spec, row_spec, hyper_spec, hyper_spec],
        out_specs=[out_spec, out_spec],
        out_shape=[
            jax.ShapeDtypeStruct((N_USERS, DFUSED), jnp.float32),
            jax.ShapeDtypeStruct((N_USERS, DFUSED), jnp.float32),
        ],
    )(ukey, uemb, ikey, iemb, uh, ih)


def _sc_body(a_hbm, b_hbm, rows_hbm, cols_hbm, eids_hbm, out_hbm,
             eids_v, u_v, i_v, a_v, b_v, sa_v, sb_v, out_v,
             sem_e0, sem_e1, sem_ui0, sem_ui1, sem_ab0, sem_ab1):
    cid = lax.axis_index("c")
    sid = lax.axis_index("s")
    wid = sid * NC + cid
    my_n = BASE_CHUNKS + jnp.where(wid < EXTRA_CHUNKS, 1, 0)
    sem_e = (sem_e0, sem_e1)
    sem_ui = (sem_ui0, sem_ui1)
    sem_ab = (sem_ab0, sem_ab1)

    def goff(j):
        return (wid + j * NW) * CHUNK

    def eids_slot(s):
        return eids_v.at[s]

    def issue_eids(j, s):
        return pltpu.async_copy(eids_hbm.at[pl.ds(goff(j), CHUNK)],
                                eids_slot(s), sem_e[s])

    def issue_ui(s):
        cu = pltpu.async_copy(rows_hbm.at[eids_slot(s)], u_v.at[s], sem_ui[s])
        ci = pltpu.async_copy(cols_hbm.at[eids_slot(s)], i_v.at[s], sem_ui[s])
        return cu, ci

    def issue_ab(s):
        ca = pltpu.async_copy(a_hbm.at[u_v.at[s]], a_v.at[s], sem_ab[s])
        cb = pltpu.async_copy(b_hbm.at[i_v.at[s]], b_v.at[s], sem_ab[s])
        return ca, cb

    def wait_eids(s):
        pltpu.make_async_copy(eids_hbm.at[pl.ds(0, CHUNK)], eids_slot(s),
                              sem_e[s]).wait()

    def wait_ui(s):
        pltpu.make_async_copy(rows_hbm.at[pl.ds(0, CHUNK)], u_v.at[s],
                              sem_ui[s]).wait()
        pltpu.make_async_copy(cols_hbm.at[pl.ds(0, CHUNK)], i_v.at[s],
                              sem_ui[s]).wait()

    def wait_ab(s):
        pltpu.make_async_copy(a_hbm.at[pl.ds(0, CHUNK)], a_v.at[s],
                              sem_ab[s]).wait()
        pltpu.make_async_copy(b_hbm.at[pl.ds(0, CHUNK)], b_v.at[s],
                              sem_ab[s]).wait()

    def compute_chunk(k, s):
        last_lane = lax.iota(jnp.int32, LANES) == (LANES - 1)

        def edge_body(e, _):
            ev = jnp.full((LANES,), e, dtype=jnp.int32)
            pa = jnp.zeros((LANES,), jnp.float32)
            pb = jnp.zeros((LANES,), jnp.float32)
            for c in range(LATDIM // LANES):
                pa = pa + (a_v[s, e, pl.ds(c * LANES, LANES)]
                           * b_v[s, e, pl.ds(c * LANES, LANES)])
            for c in range(LATDIM // LANES, DFUSED // LANES):
                pb = pb + (a_v[s, e, pl.ds(c * LANES, LANES)]
                           * b_v[s, e, pl.ds(c * LANES, LANES)])
            plsc.store_scatter(sa_v, [ev], plsc.cumsum(pa), mask=last_lane)
            plsc.store_scatter(sb_v, [ev], plsc.cumsum(pb), mask=last_lane)
            return _

        lax.fori_loop(0, CHUNK, edge_body, None, unroll=8)
        for g in range(CHUNK // LANES):
            sl = pl.ds(g * LANES, LANES)
            score = 1.0 / (1.0 + jnp.exp(-sa_v[sl]))
            out_v[sl] = jnp.abs(score - sb_v[sl])
        pltpu.sync_copy(out_v, out_hbm.at[pl.ds(goff(k), CHUNK)])

    def step(k, s):
        t = 1 - s
        wait_ab(s)

        @pl.when(k + 2 < my_n)
        def _prefetch_ui():
            wait_eids(s)
            issue_ui(s)

        @pl.when(k + 1 < my_n)
        def _start_ab():
            wait_ui(t)
            issue_ab(t)

        @pl.when(k + 3 < my_n)
        def _prefetch_eids():
            issue_eids(k + 3, t)

        compute_chunk(k, s)

    issue_eids(0, 0).wait()
    issue_ui(0)
    issue_eids(1, 1).wait()
    wait_ui(0)
    issue_ab(0)
    issue_ui(1)
    issue_eids(2, 0)

    def pair_body(p, _):
        k0 = 2 * p
        step(k0, 0)

        @pl.when(k0 + 1 < my_n)
        def _odd():
            step(k0 + 1, 1)

        return _

    lax.fori_loop(0, (my_n + 1) // 2, pair_body, None)


def _edge_scores(a_tab, b_tab, rows, cols, edgeids):
    mesh = plsc.VectorSubcoreMesh(core_axis_name="c", subcore_axis_name="s",
                                  num_cores=NC, num_subcores=NS)
    f = pl.kernel(
        _sc_body,
        out_type=jax.ShapeDtypeStruct((E_TOTAL,), jnp.float32),
        mesh=mesh,
        compiler_params=pltpu.CompilerParams(needs_layout_passes=False),
        scratch_types=[
            pltpu.VMEM((2, CHUNK), jnp.int32),
            pltpu.VMEM((2, CHUNK), jnp.int32),
            pltpu.VMEM((2, CHUNK), jnp.int32),
            pltpu.VMEM((2, CHUNK, DFUSED), jnp.float32),
            pltpu.VMEM((2, CHUNK, DFUSED), jnp.float32),
            pltpu.VMEM((CHUNK,), jnp.float32),
            pltpu.VMEM((CHUNK,), jnp.float32),
            pltpu.VMEM((CHUNK,), jnp.float32),
            pltpu.SemaphoreType.DMA,
            pltpu.SemaphoreType.DMA,
            pltpu.SemaphoreType.DMA,
            pltpu.SemaphoreType.DMA,
            pltpu.SemaphoreType.DMA,
            pltpu.SemaphoreType.DMA,
        ],
    )
    return f(a_tab, b_tab, rows, cols, edgeids)


def kernel(ui_uKey, ui_iKey, uEmbeds, iEmbeds, ui_uHyper, ui_iHyper,
           rows, cols, edgeids):
    ukey = jnp.transpose(ui_uKey, (1, 0, 2)).reshape(-1, LATDIM)
    ikey = jnp.transpose(ui_iKey, (1, 0, 2)).reshape(-1, LATDIM)
    a_tab, b_tab = _prep_tables(ukey, uEmbeds, ikey, iEmbeds,
                                ui_uHyper, ui_iHyper)
    return _edge_scores(a_tab, b_tab, rows, cols, edgeids)
